# baseline (speedup 1.0000x reference)
import functools

import jax
import jax.numpy as jnp
from jax import lax
from jax.experimental import pallas as pl
from jax.experimental.pallas import tpu as pltpu
from jax.experimental.pallas import tpu_sc as plsc

_NUM_QUERIES = 300
_HIDDEN_DIM = 256


def _expand_sc(table, batch_size):
    info = plsc.get_sparse_core_info()
    num_workers = info.num_cores * info.num_subcores
    b_per_w = batch_size // num_workers

    mesh = plsc.VectorSubcoreMesh(core_axis_name="c", subcore_axis_name="s")

    @functools.partial(
        pl.kernel,
        mesh=mesh,
        out_type=jax.ShapeDtypeStruct(
            (batch_size, _NUM_QUERIES, _HIDDEN_DIM), jnp.float32
        ),
        scratch_types=[
            pltpu.VMEM((_NUM_QUERIES, _HIDDEN_DIM), jnp.float32),
            pltpu.SemaphoreType.DMA,
            pltpu.SemaphoreType.DMA,
        ],
    )
    def k(table_hbm, out_hbm, tab_v, sem0, sem1):
        wid = lax.axis_index("s") * info.num_cores + lax.axis_index("c")
        base = wid * b_per_w
        pltpu.sync_copy(table_hbm, tab_v)
        c0 = pltpu.async_copy(tab_v, out_hbm.at[base], sem0)
        c1 = pltpu.async_copy(tab_v, out_hbm.at[base + 1], sem1)
        c0.wait()
        c1.wait()

    return k(table)


def kernel(batch_ref, table):
    return _expand_sc(table, batch_ref.shape[0])


# speedup vs baseline: 1.7763x; 1.7763x over previous
import functools

import jax
import jax.numpy as jnp
from jax import lax
from jax.experimental import pallas as pl
from jax.experimental.pallas import tpu as pltpu
from jax.experimental.pallas import tpu_sc as plsc

_TILE = 8


def _expand_sc(table, batch_size):
    info = plsc.get_sparse_core_info()
    nw = info.num_cores * info.num_subcores
    nqueries, d = table.shape
    ngroups = batch_size // _TILE
    nq = -(-nqueries // nw)
    lbuf = 3 * _TILE
    pad_rows = -(-nqueries // _TILE) * _TILE

    mesh = plsc.VectorSubcoreMesh(core_axis_name="c", subcore_axis_name="s")

    @functools.partial(
        pl.kernel,
        mesh=mesh,
        out_type=jax.ShapeDtypeStruct(
            (nqueries, ngroups, _TILE, d), jnp.float32
        ),
        scratch_types=[
            pltpu.VMEM((lbuf, d), jnp.float32),
            pltpu.VMEM((nq, _TILE, d), jnp.float32),
            pltpu.SemaphoreType.DMA,
        ],
    )
    def k(table_hbm, out_hbm, tab_v, rep_v, sem):
        wid = lax.axis_index("s") * info.num_cores + lax.axis_index("c")
        q0 = (wid * nqueries) // nw
        q0a = lax.min((q0 // _TILE) * _TILE, pad_rows - lbuf)
        off = q0 - q0a
        pltpu.sync_copy(table_hbm.at[pl.ds(q0a, lbuf)], tab_v)
        for j in range(nq):
            for i in range(d // 16):
                v = tab_v[off + j, pl.ds(i * 16, 16)]
                for t in range(_TILE):
                    rep_v[j, t, pl.ds(i * 16, 16)] = v
        copies = [
            pltpu.async_copy(rep_v, out_hbm.at[pl.ds(q0, nq), g], sem)
            for g in range(ngroups)
        ]
        for c in copies:
            c.wait()

    return k(table)


def kernel(batch_ref, table):
    batch_size = batch_ref.shape[0]
    out4 = _expand_sc(table, batch_size)
    out3 = out4.reshape(table.shape[0], batch_size, table.shape[1])
    return jnp.transpose(out3, (1, 0, 2))


# speedup vs baseline: 1.8153x; 1.0219x over previous
import functools

import jax
import jax.numpy as jnp
from jax import lax
from jax.experimental import pallas as pl
from jax.experimental.pallas import tpu as pltpu
from jax.experimental.pallas import tpu_sc as plsc

_TILE = 8


def _expand_sc(table, batch_size):
    info = plsc.get_sparse_core_info()
    nw = info.num_cores * info.num_subcores
    nqueries, d = table.shape
    ngroups = batch_size // _TILE
    nq = -(-nqueries // nw)
    lbuf = 3 * _TILE
    pad_rows = -(-nqueries // _TILE) * _TILE

    mesh = plsc.VectorSubcoreMesh(core_axis_name="c", subcore_axis_name="s")

    @functools.partial(
        pl.kernel,
        mesh=mesh,
        out_type=jax.ShapeDtypeStruct(
            (nqueries, ngroups, _TILE, d), jnp.float32
        ),
        scratch_types=[
            pltpu.VMEM((lbuf, d), jnp.float32),
            pltpu.VMEM((nq, _TILE, d), jnp.float32),
            pltpu.SemaphoreType.DMA,
        ],
    )
    def k(table_hbm, out_hbm, tab_v, rep_v, sem):
        wid = lax.axis_index("s") * info.num_cores + lax.axis_index("c")
        q0 = (wid * nqueries) // nw
        q0a = lax.min((q0 // _TILE) * _TILE, pad_rows - lbuf)
        off = q0 - q0a
        pltpu.sync_copy(table_hbm.at[pl.ds(q0a, lbuf)], tab_v)
        half = nq // 2
        copies = []
        for h, hn in ((0, half), (half, nq - half)):
            for j in range(h, h + hn):
                for i in range(d // 16):
                    v = tab_v[off + j, pl.ds(i * 16, 16)]
                    for t in range(_TILE):
                        rep_v[j, t, pl.ds(i * 16, 16)] = v
            copies += [
                pltpu.async_copy(
                    rep_v.at[pl.ds(h, hn)],
                    out_hbm.at[pl.ds(q0 + h, hn), g],
                    sem,
                )
                for g in range(ngroups)
            ]
        for c in copies:
            c.wait()

    return k(table)


def kernel(batch_ref, table):
    batch_size = batch_ref.shape[0]
    out4 = _expand_sc(table, batch_size)
    out3 = out4.reshape(table.shape[0], batch_size, table.shape[1])
    return jnp.transpose(out3, (1, 0, 2))


# speedup vs baseline: 1.8378x; 1.0124x over previous
import functools

import jax
import jax.numpy as jnp
from jax import lax
from jax.experimental import pallas as pl
from jax.experimental.pallas import tpu as pltpu
from jax.experimental.pallas import tpu_sc as plsc

_TILE = 8


def _expand_sc(table, batch_size):
    info = plsc.get_sparse_core_info()
    nw = info.num_cores * info.num_subcores
    nqueries, d = table.shape
    ngroups = batch_size // _TILE
    nq = -(-nqueries // nw)
    lbuf = 3 * _TILE
    pad_rows = -(-nqueries // _TILE) * _TILE

    mesh = plsc.VectorSubcoreMesh(core_axis_name="c", subcore_axis_name="s")

    @functools.partial(
        pl.kernel,
        mesh=mesh,
        out_type=jax.ShapeDtypeStruct(
            (nqueries, ngroups, _TILE, d), jnp.float32
        ),
        scratch_types=[
            pltpu.VMEM((lbuf, d), jnp.float32),
            pltpu.VMEM((nq, _TILE, d), jnp.float32),
            pltpu.SemaphoreType.DMA,
        ],
    )
    def k(table_hbm, out_hbm, tab_v, rep_v, sem):
        wid = lax.axis_index("s") * info.num_cores + lax.axis_index("c")
        q0 = (wid * nqueries) // nw
        q0a = lax.min((q0 // _TILE) * _TILE, pad_rows - lbuf)
        off = q0 - q0a
        pltpu.sync_copy(table_hbm.at[pl.ds(q0a, lbuf)], tab_v)
        chunks = []
        pos = 0
        for cn in (2, 2, 3, 3):
            chunks.append((pos, cn))
            pos += cn
        assert pos == nq
        copies = []
        for h, hn in chunks:
            for j in range(h, h + hn):
                for i in range(d // 16):
                    v = tab_v[off + j, pl.ds(i * 16, 16)]
                    for t in range(_TILE):
                        rep_v[j, t, pl.ds(i * 16, 16)] = v
            copies += [
                pltpu.async_copy(
                    rep_v.at[pl.ds(h, hn)],
                    out_hbm.at[pl.ds(q0 + h, hn), g],
                    sem,
                )
                for g in range(ngroups)
            ]
        for c in copies:
            c.wait()

    return k(table)


def kernel(batch_ref, table):
    batch_size = batch_ref.shape[0]
    out4 = _expand_sc(table, batch_size)
    out3 = out4.reshape(table.shape[0], batch_size, table.shape[1])
    return jnp.transpose(out3, (1, 0, 2))


# speedup vs baseline: 2.6905x; 1.4639x over previous
import jax
import jax.numpy as jnp
from jax.experimental import pallas as pl


def _expand_tc(table, batch_size):
    nqueries, d = table.shape
    bq = 10
    t3 = table.reshape(nqueries // bq, bq, d)

    def body(tab_ref, out_ref):
        out_ref[...] = jnp.broadcast_to(
            tab_ref[0][:, None, :], (bq, batch_size, d)
        )

    return pl.pallas_call(
        body,
        grid=(nqueries // bq,),
        in_specs=[pl.BlockSpec((1, bq, d), lambda i: (i, 0, 0))],
        out_specs=pl.BlockSpec((bq, batch_size, d), lambda i: (i, 0, 0)),
        out_shape=jax.ShapeDtypeStruct(
            (nqueries, batch_size, d), jnp.float32
        ),
    )(t3)


def kernel(batch_ref, table):
    out_t = _expand_tc(table, batch_ref.shape[0])
    return jnp.transpose(out_t, (1, 0, 2))


# speedup vs baseline: 4.7745x; 1.7746x over previous
import jax
import jax.numpy as jnp
from jax.experimental import pallas as pl


def _expand_tc(table, batch_size):
    nqueries, d = table.shape
    bq = 30
    t3 = table.reshape(nqueries // bq, bq, d)

    def body(tab_ref, out_ref):
        out_ref[...] = jnp.broadcast_to(
            tab_ref[0][:, None, :], (bq, batch_size, d)
        )

    return pl.pallas_call(
        body,
        grid=(nqueries // bq,),
        in_specs=[pl.BlockSpec((1, bq, d), lambda i: (i, 0, 0))],
        out_specs=pl.BlockSpec((bq, batch_size, d), lambda i: (i, 0, 0)),
        out_shape=jax.ShapeDtypeStruct(
            (nqueries, batch_size, d), jnp.float32
        ),
    )(t3)


def kernel(batch_ref, table):
    out_t = _expand_tc(table, batch_ref.shape[0])
    return jnp.transpose(out_t, (1, 0, 2))


# speedup vs baseline: 5.7536x; 1.2051x over previous
import jax
import jax.numpy as jnp
from jax.experimental import pallas as pl


def _expand_tc(table, batch_size):
    nqueries, d = table.shape
    bq = 75
    t3 = table.reshape(nqueries // bq, bq, d)

    def body(tab_ref, out_ref):
        out_ref[...] = jnp.broadcast_to(
            tab_ref[0][:, None, :], (bq, batch_size, d)
        )

    return pl.pallas_call(
        body,
        grid=(nqueries // bq,),
        in_specs=[pl.BlockSpec((1, bq, d), lambda i: (i, 0, 0))],
        out_specs=pl.BlockSpec((bq, batch_size, d), lambda i: (i, 0, 0)),
        out_shape=jax.ShapeDtypeStruct(
            (nqueries, batch_size, d), jnp.float32
        ),
    )(t3)


def kernel(batch_ref, table):
    out_t = _expand_tc(table, batch_ref.shape[0])
    return jnp.transpose(out_t, (1, 0, 2))
